# baseline (speedup 1.0000x reference)
import functools

import jax
import jax.numpy as jnp
from jax import lax
from jax.experimental import pallas as pl
from jax.experimental.pallas import tpu as pltpu
from jax.experimental.pallas import tpu_sc as plsc

VOCAB = 1024
D = 128
CHUNK = 128


@functools.cache
def _make_sc_gather(N):
    info = plsc.get_sparse_core_info()
    NC, NS = info.num_cores, info.num_subcores
    NW = NC * NS
    n_per_w = N // NW
    n_chunks = n_per_w // CHUNK
    assert n_per_w * NW == N and n_chunks * CHUNK == n_per_w

    mesh = plsc.VectorSubcoreMesh(core_axis_name="c", subcore_axis_name="s")

    NBUF = 2
    n_pairs = n_chunks // 2
    assert n_pairs % NBUF == 0 and VOCAB % NS == 0

    @functools.partial(
        pl.kernel,
        mesh=mesh,
        out_type=jax.ShapeDtypeStruct((N, D), jnp.float32),
        scratch_types=[
            pltpu.MemorySpace.VMEM_SHARED((VOCAB, D), jnp.float32),
            pltpu.VMEM((n_chunks, CHUNK), jnp.int32),
        ]
        + [pltpu.VMEM((2 * CHUNK, D), jnp.float32)] * NBUF
        + [pltpu.SemaphoreType.DMA] * (3 * NBUF + 2),
    )
    def gather_kernel(x_hbm, table_hbm, out_hbm, table_sh, idx_v, *rest):
        bufs = rest[:NBUF]
        gsem = rest[NBUF:3 * NBUF]
        osem = rest[3 * NBUF:3 * NBUF + NBUF]
        tsem, isem = rest[3 * NBUF + NBUF:]
        sid = lax.axis_index("s")
        wid = sid * NC + lax.axis_index("c")
        base = wid * n_per_w

        rows = VOCAB // NS
        tab_copy = pltpu.make_async_copy(
            table_hbm.at[pl.ds(sid * rows, rows)],
            table_sh.at[pl.ds(sid * rows, rows)], tsem)
        idx_copy = pltpu.make_async_copy(x_hbm.at[wid], idx_v, isem)
        tab_copy.start()
        idx_copy.start()
        idx_copy.wait()

        def gather(p, b):
            for h in range(2):
                pltpu.async_copy(
                    table_sh.at[idx_v.at[2 * p + h]],
                    bufs[b].at[pl.ds(h * CHUNK, CHUNK)], gsem[2 * b + h])

        def gather_wait(p, b):
            for h in range(2):
                pltpu.make_async_copy(
                    table_sh.at[idx_v.at[2 * p + h]],
                    bufs[b].at[pl.ds(h * CHUNK, CHUNK)],
                    gsem[2 * b + h]).wait()

        def write(p, b):
            pltpu.async_copy(
                bufs[b], out_hbm.at[pl.ds(base + p * 2 * CHUNK, 2 * CHUNK)],
                osem[b])

        def write_wait(p, b):
            pltpu.make_async_copy(
                bufs[b], out_hbm.at[pl.ds(base + p * 2 * CHUNK, 2 * CHUNK)],
                osem[b]).wait()

        for h in range(2):
            pltpu.async_copy(table_hbm.at[idx_v.at[h]],
                             bufs[0].at[pl.ds(h * CHUNK, CHUNK)], gsem[h])
        tab_copy.wait()
        plsc.subcore_barrier()
        for b in range(1, NBUF):
            gather(b, b)

        def body(i, carry):
            for b in range(NBUF):
                p = NBUF * i + b
                gather_wait(p, b)
                write(p, b)

                @pl.when(p + NBUF < n_pairs)
                def _():
                    write_wait(p, b)
                    gather(p + NBUF, b)

            return carry

        lax.fori_loop(0, n_pairs // NBUF, body, 0)
        for b in range(NBUF):
            write_wait(n_pairs - NBUF + b, b)

    return gather_kernel


def kernel(x, table):
    B, L = x.shape
    N = B * L
    info = plsc.get_sparse_core_info()
    NW = info.num_cores * info.num_subcores
    xf = x.astype(jnp.int32).reshape(NW, N // (NW * CHUNK), CHUNK)
    tpad = jnp.zeros((VOCAB, D), table.dtype).at[:table.shape[0]].set(table)
    out = _make_sc_gather(N)(xf, tpad)
    return out.reshape(B, L, D)[:, None]


# speedup vs baseline: 1.0213x; 1.0213x over previous
import functools

import jax
import jax.numpy as jnp
from jax import lax
from jax.experimental import pallas as pl
from jax.experimental.pallas import tpu as pltpu
from jax.experimental.pallas import tpu_sc as plsc

VOCAB = 1024
D = 128
CHUNK = 128


@functools.cache
def _make_sc_gather(N):
    info = plsc.get_sparse_core_info()
    NC, NS = info.num_cores, info.num_subcores
    NW = NC * NS
    n_per_w = N // NW
    n_chunks = n_per_w // CHUNK
    assert n_per_w * NW == N and n_chunks * CHUNK == n_per_w

    mesh = plsc.VectorSubcoreMesh(core_axis_name="c", subcore_axis_name="s")

    NBUF = 2
    n_pairs = n_chunks // 2
    assert n_pairs % NBUF == 0 and VOCAB % NS == 0

    @functools.partial(
        pl.kernel,
        mesh=mesh,
        out_type=jax.ShapeDtypeStruct((N, D), jnp.float32),
        scratch_types=[
            pltpu.MemorySpace.VMEM_SHARED((VOCAB, D), jnp.float32),
            pltpu.VMEM((n_chunks, CHUNK), jnp.int32),
        ]
        + [pltpu.VMEM((2 * CHUNK, D), jnp.float32)] * NBUF
        + [pltpu.SemaphoreType.DMA] * (3 * NBUF + 2),
    )
    def gather_kernel(x_hbm, table_hbm, out_hbm, table_sh, idx_v, *rest):
        bufs = rest[:NBUF]
        gsem = rest[NBUF:3 * NBUF]
        osem = rest[3 * NBUF:3 * NBUF + NBUF]
        tsem, isem = rest[3 * NBUF + NBUF:]
        sid = lax.axis_index("s")
        wid = sid * NC + lax.axis_index("c")
        base = wid * n_per_w

        rows = VOCAB // NS
        tab_copy = pltpu.make_async_copy(
            table_hbm.at[pl.ds(sid * rows, rows)],
            table_sh.at[pl.ds(sid * rows, rows)], tsem)
        idx_copy = pltpu.make_async_copy(x_hbm.at[wid], idx_v, isem)
        tab_copy.start()
        idx_copy.start()
        tab_copy.wait()
        idx_copy.wait()
        plsc.subcore_barrier()

        def gather(p, b):
            for h in range(2):
                pltpu.async_copy(
                    table_sh.at[idx_v.at[2 * p + h]],
                    bufs[b].at[pl.ds(h * CHUNK, CHUNK)], gsem[2 * b + h])

        def gather_wait(p, b):
            for h in range(2):
                pltpu.make_async_copy(
                    table_sh.at[idx_v.at[2 * p + h]],
                    bufs[b].at[pl.ds(h * CHUNK, CHUNK)],
                    gsem[2 * b + h]).wait()

        def write(p, b):
            pltpu.async_copy(
                bufs[b], out_hbm.at[pl.ds(base + p * 2 * CHUNK, 2 * CHUNK)],
                osem[b])

        def write_wait(p, b):
            pltpu.make_async_copy(
                bufs[b], out_hbm.at[pl.ds(base + p * 2 * CHUNK, 2 * CHUNK)],
                osem[b]).wait()

        for b in range(NBUF):
            gather(b, b)

        def body(i, carry):
            for b in range(NBUF):
                p = NBUF * i + b
                gather_wait(p, b)
                write(p, b)

                @pl.when(p + NBUF < n_pairs)
                def _():
                    write_wait(p, b)
                    gather(p + NBUF, b)

            return carry

        lax.fori_loop(0, n_pairs // NBUF, body, 0)
        for b in range(NBUF):
            write_wait(n_pairs - NBUF + b, b)

    return gather_kernel


def kernel(x, table):
    B, L = x.shape
    N = B * L
    info = plsc.get_sparse_core_info()
    NW = info.num_cores * info.num_subcores
    xf = x.astype(jnp.int32).reshape(NW, N // (NW * CHUNK), CHUNK)
    tpad = jnp.zeros((VOCAB, D), table.dtype).at[:table.shape[0]].set(table)
    out = _make_sc_gather(N)(xf, tpad)
    return out.reshape(B, L, D)[:, None]
